# baseline (speedup 1.0000x reference)
import functools

import jax
import jax.numpy as jnp
from jax import lax
from jax.experimental import pallas as pl
from jax.experimental.pallas import tpu as pltpu
from jax.experimental.pallas import tpu_sc as plsc

UNK = 0
IGNORE = -100
EPS = 1e-20
LN2 = 0.6931471805599453
SQRT2 = 1.4142135623730951


def _vlog(x):
    bits = lax.bitcast_convert_type(x, jnp.int32)
    e = ((bits >> 23) & 0xFF) - 127
    m = lax.bitcast_convert_type((bits & 0x007FFFFF) | 0x3F800000, jnp.float32)
    big = m >= SQRT2
    m = jnp.where(big, m * 0.5, m)
    e = jnp.where(big, e + 1, e)
    s = (m - 1.0) / (m + 1.0)
    s2 = s * s
    p = 2.0 * s * (1.0 + s2 * (1.0 / 3.0 + s2 * (0.2 + s2 * (1.0 / 7.0))))
    return e.astype(jnp.float32) * LN2 + p


def kernel(scores, align_src, align_conv, target):
    planes, n, vocab = scores.shape
    rows_per_plane = vocab * n // 128
    scores_t = jnp.transpose(scores, (0, 2, 1))
    align_src = align_src.astype(jnp.int32)
    align_conv = align_conv.astype(jnp.int32)
    target = target.astype(jnp.int32)

    info = plsc.get_sparse_core_info()
    nc = 2
    nw = nc * info.num_subcores
    bw = n // nw
    nch = bw // 16

    mesh = plsc.VectorSubcoreMesh(core_axis_name="c", subcore_axis_name="s",
                                  num_cores=nc)

    @functools.partial(
        pl.kernel,
        mesh=mesh,
        compiler_params=pltpu.CompilerParams(needs_layout_passes=False),
        out_type=jax.ShapeDtypeStruct((n,), jnp.float32),
        scratch_types=[
            pltpu.VMEM((bw,), jnp.int32),
            pltpu.VMEM((bw,), jnp.int32),
            pltpu.VMEM((bw,), jnp.int32),
            pltpu.VMEM((3 * bw,), jnp.int32),
            pltpu.VMEM((3 * bw, 128), jnp.float32),
            pltpu.VMEM((bw,), jnp.float32),
            pltpu.SemaphoreType.DMA,
            pltpu.SemaphoreType.DMA,
        ],
    )
    def _k(scores_hbm, tgt_hbm, src_hbm, conv_hbm, out_hbm,
           tgt_v, src_v, conv_v, idx_v, win_v, out_v, sem_i, sem_w):
        wid = lax.axis_index("s") * nc + lax.axis_index("c")
        base = wid * bw
        tab = scores_hbm.reshape(planes * vocab, n)
        cp_t = pltpu.make_async_copy(tgt_hbm.at[pl.ds(base, bw)], tgt_v, sem_i)
        cp_s = pltpu.make_async_copy(src_hbm.at[pl.ds(base, bw)], src_v, sem_i)
        cp_c = pltpu.make_async_copy(conv_hbm.at[pl.ds(base, bw)], conv_v, sem_i)
        cp_t.start(); cp_s.start(); cp_c.start()
        cp_t.wait(); cp_s.wait(); cp_c.wait()
        lanes = lax.iota(jnp.int32, 16)

        def _build(jj, _):
            o = jj * 16
            idx_v[pl.ds(o, 16)] = tgt_v[pl.ds(o, 16)]
            idx_v[pl.ds(bw + o, 16)] = vocab + src_v[pl.ds(o, 16)]
            idx_v[pl.ds(2 * bw + o, 16)] = 2 * vocab + conv_v[pl.ds(o, 16)]
            return 0

        lax.fori_loop(0, nch, _build, 0)
        rs = pl.multiple_of((base >> 7) << 7, 128)
        pltpu.async_copy(tab.at[idx_v, pl.ds(rs, 128)], win_v, sem_w).wait()

        def _extract(jj, _):
            o = jj * 16
            j16 = o + lanes
            rloc = (base & 127) + j16
            tgt = tgt_v[pl.ds(o, 16)]
            src = src_v[pl.ds(o, 16)]
            conv = conv_v[pl.ds(o, 16)]
            v = plsc.load_gather(win_v, [j16, rloc])
            c = plsc.load_gather(win_v, [bw + j16, rloc])
            cc = plsc.load_gather(win_v, [2 * bw + j16, rloc])
            src_unk = src == UNK
            conv_unk = conv == UNK
            ct = jnp.where(src_unk, 0.0, c) + EPS
            ccv = jnp.where(conv_unk, 0.0, cc) + EPS
            non_copy = (src_unk & conv_unk) | (tgt != UNK)
            probs = ct + ccv + jnp.where(non_copy, v, 0.0)
            loss = -_vlog(probs)
            out_v[pl.ds(o, 16)] = jnp.where(tgt == IGNORE, 0.0, loss)
            return 0

        lax.fori_loop(0, nch, _extract, 0)
        pltpu.sync_copy(out_v, out_hbm.at[pl.ds(base, bw)])

    return _k(scores_t, target, align_src, align_conv)


# speedup vs baseline: 1.0101x; 1.0101x over previous
import functools

import jax
import jax.numpy as jnp
from jax import lax
from jax.experimental import pallas as pl
from jax.experimental.pallas import tpu as pltpu
from jax.experimental.pallas import tpu_sc as plsc

UNK = 0
IGNORE = -100
EPS = 1e-20
LN2 = 0.6931471805599453
SQRT2 = 1.4142135623730951


def _vlog(x):
    bits = lax.bitcast_convert_type(x, jnp.int32)
    e = ((bits >> 23) & 0xFF) - 127
    m = lax.bitcast_convert_type((bits & 0x007FFFFF) | 0x3F800000, jnp.float32)
    big = m >= SQRT2
    m = jnp.where(big, m * 0.5, m)
    e = jnp.where(big, e + 1, e)
    s = (m - 1.0) / (m + 1.0)
    s2 = s * s
    p = 2.0 * s * (1.0 + s2 * (1.0 / 3.0 + s2 * (0.2 + s2 * (1.0 / 7.0))))
    return e.astype(jnp.float32) * LN2 + p


def kernel(scores, align_src, align_conv, target):
    planes, n, vocab = scores.shape
    rows_per_plane = vocab * n // 128
    scores_t = jnp.transpose(scores, (0, 2, 1))
    align_src = align_src.astype(jnp.int32)
    align_conv = align_conv.astype(jnp.int32)
    target = target.astype(jnp.int32)

    info = plsc.get_sparse_core_info()
    nc = 1
    nw = nc * info.num_subcores
    bw = n // nw
    nch = bw // 16

    mesh = plsc.VectorSubcoreMesh(core_axis_name="c", subcore_axis_name="s",
                                  num_cores=nc)

    @functools.partial(
        pl.kernel,
        mesh=mesh,
        compiler_params=pltpu.CompilerParams(needs_layout_passes=False),
        out_type=jax.ShapeDtypeStruct((n,), jnp.float32),
        scratch_types=[
            pltpu.VMEM((bw,), jnp.int32),
            pltpu.VMEM((bw,), jnp.int32),
            pltpu.VMEM((bw,), jnp.int32),
            pltpu.VMEM((3 * bw,), jnp.int32),
            pltpu.VMEM((3 * bw, 128), jnp.float32),
            pltpu.VMEM((bw,), jnp.float32),
            pltpu.SemaphoreType.DMA,
            pltpu.SemaphoreType.DMA,
        ],
    )
    def _k(scores_hbm, tgt_hbm, src_hbm, conv_hbm, out_hbm,
           tgt_v, src_v, conv_v, idx_v, win_v, out_v, sem_i, sem_w):
        wid = lax.axis_index("s") * nc + lax.axis_index("c")
        base = wid * bw
        tab = scores_hbm.reshape(planes * vocab, n)
        cp_t = pltpu.make_async_copy(tgt_hbm.at[pl.ds(base, bw)], tgt_v, sem_i)
        cp_s = pltpu.make_async_copy(src_hbm.at[pl.ds(base, bw)], src_v, sem_i)
        cp_c = pltpu.make_async_copy(conv_hbm.at[pl.ds(base, bw)], conv_v, sem_i)
        cp_t.start(); cp_s.start(); cp_c.start()
        cp_t.wait(); cp_s.wait(); cp_c.wait()
        lanes = lax.iota(jnp.int32, 16)

        def _build(jj, _):
            o = jj * 16
            idx_v[pl.ds(o, 16)] = tgt_v[pl.ds(o, 16)]
            idx_v[pl.ds(bw + o, 16)] = vocab + src_v[pl.ds(o, 16)]
            idx_v[pl.ds(2 * bw + o, 16)] = 2 * vocab + conv_v[pl.ds(o, 16)]
            return 0

        lax.fori_loop(0, nch, _build, 0)
        rs = pl.multiple_of((base >> 7) << 7, 128)
        pltpu.async_copy(tab.at[idx_v, pl.ds(rs, 128)], win_v, sem_w).wait()

        def _extract(jj, _):
            o = jj * 16
            j16 = o + lanes
            rloc = (base & 127) + j16
            tgt = tgt_v[pl.ds(o, 16)]
            src = src_v[pl.ds(o, 16)]
            conv = conv_v[pl.ds(o, 16)]
            v = plsc.load_gather(win_v, [j16, rloc])
            c = plsc.load_gather(win_v, [bw + j16, rloc])
            cc = plsc.load_gather(win_v, [2 * bw + j16, rloc])
            src_unk = src == UNK
            conv_unk = conv == UNK
            ct = jnp.where(src_unk, 0.0, c) + EPS
            ccv = jnp.where(conv_unk, 0.0, cc) + EPS
            non_copy = (src_unk & conv_unk) | (tgt != UNK)
            probs = ct + ccv + jnp.where(non_copy, v, 0.0)
            loss = -_vlog(probs)
            out_v[pl.ds(o, 16)] = jnp.where(tgt == IGNORE, 0.0, loss)
            return 0

        lax.fori_loop(0, nch, _extract, 0)
        pltpu.sync_copy(out_v, out_hbm.at[pl.ds(base, bw)])

    return _k(scores_t, target, align_src, align_conv)
